# Initial kernel scaffold; baseline (speedup 1.0000x reference)
import jax
import jax.numpy as jnp
from jax.experimental import pallas as pl


def kernel(ids, table):
    raise NotImplementedError("write your pallas kernel here")


# speedup vs baseline: 4.9458x; 4.9458x over previous
import functools

import jax
import jax.numpy as jnp
from jax import lax
from jax.experimental import pallas as pl
from jax.experimental.pallas import tpu as pltpu
from jax.experimental.pallas import tpu_sc as plsc

_NW = 32
_C = 2048


@functools.lru_cache(maxsize=None)
def _make_gather(N, V, D):
    per_w = N // _NW
    n_chunks = per_w // _C
    mesh = plsc.VectorSubcoreMesh(core_axis_name="c", subcore_axis_name="s")

    @functools.partial(
        pl.kernel,
        out_type=jax.ShapeDtypeStruct((N, D), jnp.float32),
        mesh=mesh,
        scratch_types=[
            pltpu.VMEM((_C,), jnp.int32),
            pltpu.VMEM((_C, D), jnp.float32),
            pltpu.SemaphoreType.DMA,
        ],
        compiler_params=pltpu.CompilerParams(use_tc_tiling_on_sc=False),
    )
    def gather_kernel(ids_hbm, table_hbm, out_hbm, idx_v, rows_v, sem):
        wid = lax.axis_index("s") * 2 + lax.axis_index("c")
        base = wid * per_w

        def body(g, carry):
            off = base + g * _C
            pltpu.sync_copy(ids_hbm.at[pl.ds(off, _C)], idx_v)
            pltpu.async_copy(table_hbm.at[idx_v], rows_v, sem).wait()
            pltpu.sync_copy(rows_v, out_hbm.at[pl.ds(off, _C)])
            return carry

        lax.fori_loop(0, n_chunks, body, 0)

    return gather_kernel


def kernel(ids, table):
    B, H = ids.shape
    V, D = table.shape
    N = B * H
    ids_flat = ids.reshape(N).astype(jnp.int32)
    out = _make_gather(N, V, D)(ids_flat, table)
    return out.reshape(B, H, D)


# speedup vs baseline: 5.0390x; 1.0188x over previous
import functools

import jax
import jax.numpy as jnp
from jax import lax
from jax.experimental import pallas as pl
from jax.experimental.pallas import tpu as pltpu
from jax.experimental.pallas import tpu_sc as plsc

_NW = 32
_C = 1600
_NBUF = 2


@functools.lru_cache(maxsize=None)
def _make_gather(N, V, D):
    per_w = N // _NW
    n_chunks = per_w // _C
    n_groups = n_chunks // _NBUF
    mesh = plsc.VectorSubcoreMesh(core_axis_name="c", subcore_axis_name="s")

    scratch = (
        [pltpu.VMEM((_C,), jnp.int32) for _ in range(_NBUF)]
        + [pltpu.VMEM((_C, D), jnp.float32) for _ in range(_NBUF)]
        + [pltpu.SemaphoreType.DMA for _ in range(3 * _NBUF)]
    )

    @functools.partial(
        pl.kernel,
        out_type=jax.ShapeDtypeStruct((N, D), jnp.float32),
        mesh=mesh,
        scratch_types=scratch,
        compiler_params=pltpu.CompilerParams(use_tc_tiling_on_sc=False),
    )
    def gather_kernel(ids_hbm, table_hbm, out_hbm, *bufs):
        idx_v = bufs[0:_NBUF]
        rows_v = bufs[_NBUF:2 * _NBUF]
        sem_i = bufs[2 * _NBUF:3 * _NBUF]
        sem_g = bufs[3 * _NBUF:4 * _NBUF]
        sem_o = bufs[4 * _NBUF:5 * _NBUF]

        wid = lax.axis_index("s") * 2 + lax.axis_index("c")
        base = wid * per_w

        for b in range(_NBUF):
            pltpu.async_copy(
                ids_hbm.at[pl.ds(base + b * _C, _C)], idx_v[b], sem_i[b])

        def group(t, carry):
            for b in range(_NBUF):
                off = base + (t * _NBUF + b) * _C
                pltpu.make_async_copy(
                    ids_hbm.at[pl.ds(off, _C)], idx_v[b], sem_i[b]).wait()

                @pl.when(t > 0)
                def _():
                    pltpu.make_async_copy(
                        rows_v[b], out_hbm.at[pl.ds(off - _NBUF * _C, _C)],
                        sem_o[b]).wait()

                pltpu.async_copy(
                    table_hbm.at[idx_v[b]], rows_v[b], sem_g[b]).wait()

                pltpu.async_copy(
                    rows_v[b], out_hbm.at[pl.ds(off, _C)], sem_o[b])

                @pl.when(t < n_groups - 1)
                def _():
                    pltpu.async_copy(
                        ids_hbm.at[pl.ds(off + _NBUF * _C, _C)],
                        idx_v[b], sem_i[b])
            return carry

        lax.fori_loop(0, n_groups, group, 0)

        for b in range(_NBUF):
            off = base + ((n_groups - 1) * _NBUF + b) * _C
            pltpu.make_async_copy(
                rows_v[b], out_hbm.at[pl.ds(off, _C)], sem_o[b]).wait()

    return gather_kernel


def kernel(ids, table):
    B, H = ids.shape
    V, D = table.shape
    N = B * H
    ids_flat = ids.reshape(N).astype(jnp.int32)
    out = _make_gather(N, V, D)(ids_flat, table)
    return out.reshape(B, H, D)
